# baseline (speedup 1.0000x reference)
import functools

import jax
import jax.numpy as jnp
from jax import lax
from jax.experimental import pallas as pl
from jax.experimental.pallas import tpu as pltpu
from jax.experimental.pallas import tpu_sc as plsc

_VOCAB = 1000000
_EMB = 16
_B = 16384
_NF = 9
_NC_FEAT = 3
_LANES = 16

_info = plsc.get_sparse_core_info()
_NW = _info.num_cores * _info.num_subcores
_BPW = _B // _NW
_CHUNKS = _BPW // _LANES

_mesh = plsc.VectorSubcoreMesh(core_axis_name="c", subcore_axis_name="s")


@functools.partial(
    pl.kernel,
    mesh=_mesh,
    out_type=jax.ShapeDtypeStruct((_B,), jnp.float32),
    compiler_params=pltpu.CompilerParams(
        needs_layout_passes=False, use_tc_tiling_on_sc=False),
    scratch_types=(
        [pltpu.VMEM((_BPW,), jnp.int32) for _ in range(_NF)]
        + [pltpu.VMEM((_NF, _BPW, _EMB), jnp.float32)]
        + [pltpu.VMEM((_BPW,), jnp.float32) for _ in range(_NF)]
        + [pltpu.VMEM((_BPW,), jnp.float32) for _ in range(_NC_FEAT)]
        + [
            pltpu.VMEM((_BPW,), jnp.float32),
            pltpu.VMEM((_LANES,), jnp.float32),
            pltpu.VMEM((_LANES, _LANES), jnp.float32),
            pltpu.SemaphoreType.DMA,
            pltpu.SemaphoreType.DMA,
        ]
    ),
)
def _fm_sc(idx_hbm, cont_hbm, w_hbm, l_hbm, bias_hbm, out_hbm, *scratch):
    idx_vs = scratch[:_NF]
    rows_v = scratch[_NF]
    lrows_vs = scratch[_NF + 1:2 * _NF + 1]
    cont_vs = scratch[2 * _NF + 1:2 * _NF + 1 + _NC_FEAT]
    out_v, bias_v, tbuf, sem, sem2 = scratch[2 * _NF + 1 + _NC_FEAT:]

    wid = lax.axis_index("s") * _info.num_cores + lax.axis_index("c")
    base = wid * _BPW

    stage = []
    for j in range(_NF):
        stage.append(pltpu.async_copy(
            idx_hbm.at[pl.ds(j * _B + base, _BPW)], idx_vs[j], sem))
    for k in range(_NC_FEAT):
        stage.append(pltpu.async_copy(
            cont_hbm.at[pl.ds(k * _B + base, _BPW)], cont_vs[k], sem))
    stage.append(pltpu.async_copy(bias_hbm, bias_v, sem))
    for c in stage:
        c.wait()

    l_view = l_hbm.at[0]
    half = _BPW // 2

    def fire(h, s):
        lo = h * half
        out = []
        for j in range(_NF):
            idx_half = idx_vs[j].at[pl.ds(lo, half)]
            out.append(pltpu.async_copy(
                w_hbm.at[idx_half], rows_v.at[j, pl.ds(lo, half)], s))
            out.append(pltpu.async_copy(
                l_view.at[idx_half], lrows_vs[j].at[pl.ds(lo, half)], s))
        return out

    first = fire(0, sem)
    second = fire(1, sem2)
    for c in first:
        c.wait()

    lane = lax.iota(jnp.int32, _LANES)
    bias_vec = bias_v[...]

    def chunk_body(c, _):
        row0 = c * _LANES
        for r in range(_LANES):
            row = row0 + r
            e = rows_v[0, row]
            s = e
            ss = e * e
            for j in range(1, _NF):
                e = rows_v[j, row]
                s = s + e
                ss = ss + e * e
            tbuf[r] = s * s - ss
        res = jnp.zeros((_LANES,), jnp.float32)
        for dd in range(_LANES):
            col = plsc.load_gather(
                tbuf, [lane, jnp.full((_LANES,), dd, jnp.int32)])
            res = res + col
        lin = bias_vec
        for j in range(_NF - _NC_FEAT):
            lin = lin + lrows_vs[j][pl.ds(row0, _LANES)]
        for k in range(_NC_FEAT):
            lin = lin + (lrows_vs[_NF - _NC_FEAT + k][pl.ds(row0, _LANES)]
                         * cont_vs[k][pl.ds(row0, _LANES)])
        z = lin + 0.5 * res
        out_v[pl.ds(row0, _LANES)] = 1.0 / (1.0 + jnp.exp(-z))
        return ()

    lax.fori_loop(0, _CHUNKS // 2, chunk_body, (), unroll=False)
    for c in second:
        c.wait()
    lax.fori_loop(_CHUNKS // 2, _CHUNKS, chunk_body, (), unroll=False)

    pltpu.sync_copy(out_v, out_hbm.at[pl.ds(base, _BPW)])


def kernel(x, W, L, bias):
    idx = x[:, :_NF].astype(jnp.int32).T.reshape(_NF * _B)
    cont = x[:, _NF:].T.reshape(_NC_FEAT * _B)
    l_pad = jnp.pad(L.T, ((0, 0), (0, 64)))
    bias16 = jnp.broadcast_to(bias, (_LANES,))
    return _fm_sc(idx, cont, W, l_pad, bias16)


# speedup vs baseline: 1.0021x; 1.0021x over previous
import functools

import jax
import jax.numpy as jnp
from jax import lax
from jax.experimental import pallas as pl
from jax.experimental.pallas import tpu as pltpu
from jax.experimental.pallas import tpu_sc as plsc

_VOCAB = 1000000
_EMB = 16
_B = 16384
_NF = 9
_NC_FEAT = 3
_LANES = 16

_info = plsc.get_sparse_core_info()
_NW = _info.num_cores * _info.num_subcores
_BPW = _B // _NW
_CHUNKS = _BPW // _LANES

_mesh = plsc.VectorSubcoreMesh(core_axis_name="c", subcore_axis_name="s")


@functools.partial(
    pl.kernel,
    mesh=_mesh,
    out_type=jax.ShapeDtypeStruct((_B,), jnp.float32),
    compiler_params=pltpu.CompilerParams(
        needs_layout_passes=False, use_tc_tiling_on_sc=False),
    scratch_types=(
        [pltpu.VMEM((_BPW,), jnp.int32) for _ in range(_NF)]
        + [pltpu.VMEM((_NF, _BPW, _EMB), jnp.float32)]
        + [pltpu.VMEM((_BPW,), jnp.float32) for _ in range(_NF)]
        + [pltpu.VMEM((_BPW,), jnp.float32) for _ in range(_NC_FEAT)]
        + [
            pltpu.VMEM((_BPW,), jnp.float32),
            pltpu.VMEM((_LANES,), jnp.float32),
            pltpu.VMEM((_LANES, _LANES), jnp.float32),
            pltpu.SemaphoreType.DMA,
        ]
    ),
)
def _fm_sc(idx_hbm, cont_hbm, w_hbm, l_hbm, bias_hbm, out_hbm, *scratch):
    idx_vs = scratch[:_NF]
    rows_v = scratch[_NF]
    lrows_vs = scratch[_NF + 1:2 * _NF + 1]
    cont_vs = scratch[2 * _NF + 1:2 * _NF + 1 + _NC_FEAT]
    out_v, bias_v, tbuf, sem = scratch[2 * _NF + 1 + _NC_FEAT:]

    wid = lax.axis_index("s") * _info.num_cores + lax.axis_index("c")
    base = wid * _BPW

    stage = []
    for j in range(_NF):
        stage.append(pltpu.async_copy(
            idx_hbm.at[pl.ds(j * _B + base, _BPW)], idx_vs[j], sem))
    for k in range(_NC_FEAT):
        stage.append(pltpu.async_copy(
            cont_hbm.at[pl.ds(k * _B + base, _BPW)], cont_vs[k], sem))
    stage.append(pltpu.async_copy(bias_hbm, bias_v, sem))
    for c in stage:
        c.wait()

    copies = []
    l_view = l_hbm.at[0]
    for j in range(_NF):
        copies.append(pltpu.async_copy(w_hbm.at[idx_vs[j]], rows_v.at[j], sem))
        copies.append(pltpu.async_copy(l_view.at[idx_vs[j]], lrows_vs[j], sem))
    for c in copies:
        c.wait()

    lane = lax.iota(jnp.int32, _LANES)
    bias_vec = bias_v[...]

    def chunk_body(c, _):
        row0 = c * _LANES
        for r in range(_LANES):
            row = row0 + r
            e = rows_v[0, row]
            s = e
            ss = e * e
            for j in range(1, _NF):
                e = rows_v[j, row]
                s = s + e
                ss = ss + e * e
            tbuf[r] = s * s - ss
        res = jnp.zeros((_LANES,), jnp.float32)
        for dd in range(_LANES):
            col = plsc.load_gather(
                tbuf, [lane, jnp.full((_LANES,), dd, jnp.int32)])
            res = res + col
        lin = bias_vec
        for j in range(_NF - _NC_FEAT):
            lin = lin + lrows_vs[j][pl.ds(row0, _LANES)]
        for k in range(_NC_FEAT):
            lin = lin + (lrows_vs[_NF - _NC_FEAT + k][pl.ds(row0, _LANES)]
                         * cont_vs[k][pl.ds(row0, _LANES)])
        z = lin + 0.5 * res
        out_v[pl.ds(row0, _LANES)] = 1.0 / (1.0 + jnp.exp(-z))
        return ()

    lax.fori_loop(0, _CHUNKS, chunk_body, (), unroll=False)

    pltpu.sync_copy(out_v, out_hbm.at[pl.ds(base, _BPW)])


def kernel(x, W, L, bias):
    idx = x[:, :_NF].astype(jnp.int32).T.reshape(_NF * _B)
    cont = x[:, _NF:].T.reshape(_NC_FEAT * _B)
    l_pad = jnp.pad(L.T, ((0, 0), (0, 64)))
    bias16 = jnp.broadcast_to(bias, (_LANES,))
    return _fm_sc(idx, cont, W, l_pad, bias16)
